# baseline (speedup 1.0000x reference)
import dataclasses
import functools

import numpy as np
import jax
import jax.numpy as jnp
from jax import lax
from jax.experimental import pallas as pl
from jax.experimental.pallas import tpu as pltpu
from jax.experimental.pallas import tpu_sc as plsc

N = 10000
E = 320000
D_FEAT = 128
D_HID = 64
N_CLASSES = 40
N_LAYERS = 8
ALPHA = 0.8

NC = 2
NS = 16
NW = NC * NS
EPT = E // NW
CHUNK = 80
NCHUNK = EPT // CHUNK
ROWS_PT = N // NS
ZROWS = 125

_mesh = plsc.VectorSubcoreMesh(core_axis_name="c", subcore_axis_name="s")

_sc_params = pltpu.CompilerParams()
if "needs_layout_passes" in pltpu.CompilerParams.__dataclass_fields__:
    _sc_params = dataclasses.replace(_sc_params, needs_layout_passes=False)
if "use_tc_tiling_on_sc" in pltpu.CompilerParams.__dataclass_fields__:
    _sc_params = dataclasses.replace(_sc_params, use_tc_tiling_on_sc=False)


@functools.partial(
    pl.kernel,
    mesh=_mesh,
    out_type=jax.ShapeDtypeStruct((NC, N, D_HID), jnp.float32),
    scratch_types=[
        pltpu.VMEM((EPT,), jnp.int32),
        pltpu.VMEM((NCHUNK, CHUNK), jnp.int32),
        pltpu.VMEM((EPT,), jnp.float32),
        pltpu.VMEM((CHUNK, D_HID), jnp.bfloat16),
        pltpu.VMEM((CHUNK, D_HID), jnp.bfloat16),
        pltpu.VMEM((CHUNK, D_HID), jnp.bfloat16),
        pltpu.VMEM((CHUNK, D_HID), jnp.float32),
        pltpu.VMEM((CHUNK, D_HID), jnp.float32),
        pltpu.VMEM((CHUNK, D_HID), jnp.float32),
        pltpu.VMEM((ZROWS, D_HID), jnp.float32),
        pltpu.VMEM_SHARED((N, D_HID), jnp.float32),
        pltpu.SemaphoreType.DMA,
        pltpu.SemaphoreType.DMA,
        pltpu.SemaphoreType.DMA,
        pltpu.SemaphoreType.DMA,
        pltpu.SemaphoreType.DMA,
        pltpu.SemaphoreType.DMA,
    ],
    compiler_params=_sc_params,
)
def _spmm_sc(h_hbm, src_hbm, dst_hbm, w_hbm, out_hbm,
             src_v, dst_v, w_v, rows0, rows1, rows2,
             rowsf0, rowsf1, rowsf2, zero_v, acc,
             gsem0, gsem1, gsem2, ssem0, ssem1, ssem2):
    cid = lax.axis_index("c")
    sid = lax.axis_index("s")
    wid = cid * NS + sid

    zvec = jnp.zeros((16,), jnp.float32)

    @pl.loop(0, ZROWS)
    def _(r):
        for c in range(D_HID // 16):
            zero_v[r, pl.ds(c * 16, 16)] = zvec

    @pl.loop(0, ROWS_PT // ZROWS)
    def _(k):
        pltpu.sync_copy(zero_v, acc.at[pl.ds(sid * ROWS_PT + k * ZROWS, ZROWS)])

    ebase = wid * EPT
    pltpu.sync_copy(src_hbm.at[pl.ds(ebase, EPT)], src_v)
    pltpu.sync_copy(dst_hbm.at[wid], dst_v)
    pltpu.sync_copy(w_hbm.at[pl.ds(ebase, EPT)], w_v)

    @pl.loop(0, EPT // 16)
    def _(k):
        sl = pl.ds(k * 16, 16)
        w_v[sl] = w_v[sl] * (1.0 - ALPHA)

    plsc.subcore_barrier()

    rows = (rows0, rows1, rows2)
    rowsf = (rowsf0, rowsf1, rowsf2)
    gsem = (gsem0, gsem1, gsem2)
    ssem = (ssem0, ssem1, ssem2)

    def start_gather(jj, b):
        pltpu.async_copy(h_hbm.at[src_v.at[pl.ds(jj * CHUNK, CHUNK)]],
                         rows[b], gsem[b])

    def wait_gather(jj, b):
        pltpu.make_async_copy(h_hbm.at[src_v.at[pl.ds(jj * CHUNK, CHUNK)]],
                              rows[b], gsem[b]).wait()

    def start_scatter(jj, b):
        pltpu.async_copy(rowsf[b], acc.at[dst_v.at[jj]], ssem[b], add=True)

    def wait_scatter(jj, b):
        pltpu.make_async_copy(rowsf[b], acc.at[dst_v.at[jj]], ssem[b]).wait()

    _lanes = lax.iota(jnp.int32, 16)

    def _scale16(fbuf, buf, w16, g):
        for l in range(16):
            e = g * 16 + l
            wb = lax.gather(
                w16, jnp.full((16, 1), l, jnp.int32),
                lax.GatherDimensionNumbers(
                    offset_dims=(), collapsed_slice_dims=(0,),
                    start_index_map=(0,)),
                (1,), mode=lax.GatherScatterMode.PROMISE_IN_BOUNDS)
            erow = jnp.full((16,), e, jnp.int32)
            for c in range(D_HID // 32):
                xi = plsc.bitcast(buf[e, pl.ds(c * 32, 32)], jnp.int32)
                ev = plsc.bitcast(lax.shift_left(xi, 16), jnp.float32)
                od = plsc.bitcast(xi & jnp.int32(-65536), jnp.float32)
                cols = _lanes * 2 + (c * 32)
                plsc.store_scatter(fbuf, [erow, cols], ev * wb)
                plsc.store_scatter(fbuf, [erow, cols + 1], od * wb)

    def multiply_cold(jj, b):
        buf = rows[b]
        fbuf = rowsf[b]
        jb = jj * CHUNK

        @pl.loop(0, CHUNK // 16)
        def _(g):
            w16 = w_v[pl.ds(jb + g * 16, 16)]
            _scale16(fbuf, buf, w16, g)

    def multiply(jj, b):
        buf = rows[b]
        fbuf = rowsf[b]
        jb = jj * CHUNK
        for g in range(CHUNK // 16):
            w16 = w_v[pl.ds(jb + g * 16, 16)]
            _scale16(fbuf, buf, w16, g)

    start_gather(0, 0)
    start_gather(1, 1)
    for jj in range(3):
        b = jj % 3
        wait_gather(jj, b)
        multiply_cold(jj, b)
        start_scatter(jj, b)
        ba = (jj + 2) % 3
        if jj >= 1:
            wait_scatter(jj - 1, ba)
        start_gather(jj + 2, ba)

    @pl.loop(3, NCHUNK - 2, step=3)
    def _(j):
        for b in range(3):
            jj = j + b
            wait_gather(jj, b)
            multiply(jj, b)
            start_scatter(jj, b)
            ba = (b + 2) % 3
            wait_scatter(jj - 1, ba)
            start_gather(jj + 2, ba)

    for jj in (NCHUNK - 2, NCHUNK - 1):
        b = jj % 3
        wait_gather(jj, b)
        multiply_cold(jj, b)
        start_scatter(jj, b)
    wait_scatter(NCHUNK - 3, (NCHUNK - 3) % 3)
    wait_scatter(NCHUNK - 2, (NCHUNK - 2) % 3)
    wait_scatter(NCHUNK - 1, (NCHUNK - 1) % 3)

    plsc.subcore_barrier()

    pltpu.sync_copy(acc.at[pl.ds(sid * ROWS_PT, ROWS_PT)],
                    out_hbm.at[cid, pl.ds(sid * ROWS_PT, ROWS_PT)])


_BLK = 2000


def _in_proj_body(x_ref, w_ref, b_ref, o_ref):
    o_ref[...] = lax.dot_general(
        x_ref[...], w_ref[...], (((1,), (1,)), ((), ())),
        preferred_element_type=jnp.float32) + b_ref[...]


def _in_proj(x, W, b):
    return pl.pallas_call(
        _in_proj_body,
        grid=(N // _BLK,),
        in_specs=[
            pl.BlockSpec((_BLK, D_FEAT), lambda i: (i, 0)),
            pl.BlockSpec((D_HID, D_FEAT), lambda i: (0, 0)),
            pl.BlockSpec((1, D_HID), lambda i: (0, 0)),
        ],
        out_specs=pl.BlockSpec((_BLK, D_HID), lambda i: (i, 0)),
        out_shape=jax.ShapeDtypeStruct((N, D_HID), jnp.float32),
    )(x, W, b)


NP = N // 2
PBLK = 1000


def _layer_body(p0_ref, p1_ref, h0_ref, w_ref, b_ref, o_ref, *, beta):
    support = p0_ref[0] + p1_ref[0] + ALPHA * h0_ref[...]
    z = lax.dot_general(support, w_ref[0], (((1,), (1,)), ((), ())),
                        preferred_element_type=jnp.float32) + b_ref[0]
    h = jnp.maximum((1.0 - beta) * support + beta * z, 0.0)
    o_ref[...] = h.astype(jnp.bfloat16)


def _layer_tc(parts2, h02, Wd, b2, ell, beta):
    return pl.pallas_call(
        functools.partial(_layer_body, beta=beta),
        grid=(NP // PBLK,),
        in_specs=[
            pl.BlockSpec((1, PBLK, 2 * D_HID), lambda i: (0, i, 0)),
            pl.BlockSpec((1, PBLK, 2 * D_HID), lambda i: (1, i, 0)),
            pl.BlockSpec((PBLK, 2 * D_HID), lambda i: (i, 0)),
            pl.BlockSpec((1, 2 * D_HID, 2 * D_HID), lambda i: (ell, 0, 0)),
            pl.BlockSpec((1, 1, 2 * D_HID), lambda i: (ell, 0, 0)),
        ],
        out_specs=pl.BlockSpec((PBLK, 2 * D_HID), lambda i: (i, 0)),
        out_shape=jax.ShapeDtypeStruct((NP, 2 * D_HID), jnp.bfloat16),
    )(parts2, parts2, h02, Wd, b2)


def _last_body(p0_ref, p1_ref, h0_ref, w_ref, b_ref, wo_ref, bo_ref, o_ref,
               *, beta):
    support = p0_ref[0] + p1_ref[0] + ALPHA * h0_ref[...]
    z = lax.dot_general(support, w_ref[0], (((1,), (1,)), ((), ())),
                        preferred_element_type=jnp.float32) + b_ref[0]
    h = jnp.maximum((1.0 - beta) * support + beta * z, 0.0)
    o_ref[...] = lax.dot_general(
        h, wo_ref[...], (((1,), (1,)), ((), ())),
        preferred_element_type=jnp.float32) + bo_ref[...]


def _last_tc(parts2, h02, Wd, b2, ell, Wo2, bo2, beta):
    return pl.pallas_call(
        functools.partial(_last_body, beta=beta),
        grid=(NP // PBLK,),
        in_specs=[
            pl.BlockSpec((1, PBLK, 2 * D_HID), lambda i: (0, i, 0)),
            pl.BlockSpec((1, PBLK, 2 * D_HID), lambda i: (1, i, 0)),
            pl.BlockSpec((PBLK, 2 * D_HID), lambda i: (i, 0)),
            pl.BlockSpec((1, 2 * D_HID, 2 * D_HID), lambda i: (ell, 0, 0)),
            pl.BlockSpec((1, 1, 2 * D_HID), lambda i: (ell, 0, 0)),
            pl.BlockSpec((2 * N_CLASSES, 2 * D_HID), lambda i: (0, 0)),
            pl.BlockSpec((1, 2 * N_CLASSES), lambda i: (0, 0)),
        ],
        out_specs=pl.BlockSpec((PBLK, 2 * N_CLASSES), lambda i: (i, 0)),
        out_shape=jax.ShapeDtypeStruct((NP, 2 * N_CLASSES), jnp.float32),
    )(parts2, parts2, h02, Wd, b2, Wo2, bo2)


def kernel(x, edge_index, edge_weight, W_in, b_in, Ws, bs, W_out, b_out):
    src = edge_index[0]
    dst3d = edge_index[1].reshape(NW, NCHUNK, CHUNK)

    Wd = jnp.zeros((N_LAYERS, 2 * D_HID, 2 * D_HID), jnp.float32)
    Wd = Wd.at[:, :D_HID, :D_HID].set(Ws).at[:, D_HID:, D_HID:].set(Ws)
    b2 = jnp.tile(bs, (1, 2)).reshape(N_LAYERS, 1, 2 * D_HID)
    Wo2 = jnp.zeros((2 * N_CLASSES, 2 * D_HID), jnp.float32)
    Wo2 = Wo2.at[:N_CLASSES, :D_HID].set(W_out).at[
        N_CLASSES:, D_HID:].set(W_out)
    bo2 = jnp.tile(b_out, 2).reshape(1, 2 * N_CLASSES)

    h = _in_proj(x, W_in, b_in.reshape(1, D_HID))
    h02 = h.reshape(NP, 2 * D_HID)
    h_sc = h.astype(jnp.bfloat16)
    out = None
    for ell in range(N_LAYERS):
        beta = float(np.log(0.5 / (ell + 1) + 1.0))
        parts = _spmm_sc(h_sc, src, dst3d, edge_weight)
        parts2 = parts.reshape(NC, NP, 2 * D_HID)
        if ell < N_LAYERS - 1:
            h2 = _layer_tc(parts2, h02, Wd, b2, ell, beta)
            h_sc = h2.reshape(N, D_HID)
        else:
            out2 = _last_tc(parts2, h02, Wd, b2, ell, Wo2, bo2, beta)
    return out2.reshape(N, N_CLASSES)


# speedup vs baseline: 3.2545x; 3.2545x over previous
import dataclasses
import functools

import numpy as np
import jax
import jax.numpy as jnp
from jax import lax
from jax.experimental import pallas as pl
from jax.experimental.pallas import tpu as pltpu
from jax.experimental.pallas import tpu_sc as plsc

N = 10000
E = 320000
D_FEAT = 128
D_HID = 64
N_CLASSES = 40
N_LAYERS = 8
ALPHA = 0.8

NC = 2
NS = 16
NW = NC * NS
EPT = E // NW
CHUNK = 80
NCHUNK = EPT // CHUNK
ROWS_PT = N // NS
ZROWS = 125

_mesh = plsc.VectorSubcoreMesh(core_axis_name="c", subcore_axis_name="s")

_sc_params = pltpu.CompilerParams()
if "needs_layout_passes" in pltpu.CompilerParams.__dataclass_fields__:
    _sc_params = dataclasses.replace(_sc_params, needs_layout_passes=False)
if "use_tc_tiling_on_sc" in pltpu.CompilerParams.__dataclass_fields__:
    _sc_params = dataclasses.replace(_sc_params, use_tc_tiling_on_sc=False)


@functools.partial(
    pl.kernel,
    mesh=_mesh,
    out_type=jax.ShapeDtypeStruct((NC, N, D_HID), jnp.float32),
    scratch_types=[
        pltpu.VMEM((EPT,), jnp.int32),
        pltpu.VMEM((NCHUNK, CHUNK), jnp.int32),
        pltpu.VMEM((EPT,), jnp.float32),
        pltpu.VMEM((CHUNK, D_HID), jnp.float32),
        pltpu.VMEM((CHUNK, D_HID), jnp.float32),
        pltpu.VMEM((CHUNK, D_HID), jnp.float32),
        pltpu.VMEM((ZROWS, D_HID), jnp.float32),
        pltpu.VMEM_SHARED((N, D_HID), jnp.float32),
        pltpu.SemaphoreType.DMA,
        pltpu.SemaphoreType.DMA,
        pltpu.SemaphoreType.DMA,
        pltpu.SemaphoreType.DMA,
        pltpu.SemaphoreType.DMA,
        pltpu.SemaphoreType.DMA,
    ],
    compiler_params=_sc_params,
)
def _spmm_sc(h_hbm, src_hbm, dst_hbm, w_hbm, out_hbm,
             src_v, dst_v, w_v, rows0, rows1, rows2, zero_v, acc,
             gsem0, gsem1, gsem2, ssem0, ssem1, ssem2):
    cid = lax.axis_index("c")
    sid = lax.axis_index("s")
    wid = cid * NS + sid

    zvec = jnp.zeros((16,), jnp.float32)

    @pl.loop(0, ZROWS)
    def _(r):
        for c in range(D_HID // 16):
            zero_v[r, pl.ds(c * 16, 16)] = zvec

    @pl.loop(0, ROWS_PT // ZROWS)
    def _(k):
        pltpu.sync_copy(zero_v, acc.at[pl.ds(sid * ROWS_PT + k * ZROWS, ZROWS)])

    ebase = wid * EPT
    pltpu.sync_copy(src_hbm.at[pl.ds(ebase, EPT)], src_v)
    pltpu.sync_copy(dst_hbm.at[wid], dst_v)
    pltpu.sync_copy(w_hbm.at[pl.ds(ebase, EPT)], w_v)

    @pl.loop(0, EPT // 16)
    def _(k):
        sl = pl.ds(k * 16, 16)
        w_v[sl] = w_v[sl] * (1.0 - ALPHA)

    plsc.subcore_barrier()

    rows = (rows0, rows1, rows2)
    gsem = (gsem0, gsem1, gsem2)
    ssem = (ssem0, ssem1, ssem2)

    def start_gather(jj, b):
        pltpu.async_copy(h_hbm.at[src_v.at[pl.ds(jj * CHUNK, CHUNK)]],
                         rows[b], gsem[b])

    def wait_gather(jj, b):
        pltpu.make_async_copy(h_hbm.at[src_v.at[pl.ds(jj * CHUNK, CHUNK)]],
                              rows[b], gsem[b]).wait()

    def start_scatter(jj, b):
        pltpu.async_copy(rows[b], acc.at[dst_v.at[jj]], ssem[b], add=True)

    def wait_scatter(jj, b):
        pltpu.make_async_copy(rows[b], acc.at[dst_v.at[jj]], ssem[b]).wait()

    def multiply(jj, b):
        buf = rows[b]
        jb = jj * CHUNK
        for g in range(CHUNK // 16):
            w16 = w_v[pl.ds(jb + g * 16, 16)]
            for l in range(16):
                e = g * 16 + l
                wb = lax.gather(
                    w16, jnp.full((16, 1), l, jnp.int32),
                    lax.GatherDimensionNumbers(
                        offset_dims=(), collapsed_slice_dims=(0,),
                        start_index_map=(0,)),
                    (1,), mode=lax.GatherScatterMode.PROMISE_IN_BOUNDS)
                for c in range(D_HID // 16):
                    sl = pl.ds(c * 16, 16)
                    buf[e, sl] = buf[e, sl] * wb

    start_gather(0, 0)
    start_gather(1, 1)
    for jj in range(3):
        b = jj % 3
        wait_gather(jj, b)
        multiply(jj, b)
        start_scatter(jj, b)
        ba = (jj + 2) % 3
        if jj >= 1:
            wait_scatter(jj - 1, ba)
        start_gather(jj + 2, ba)

    @pl.loop(3, NCHUNK - 2, step=3)
    def _(j):
        for b in range(3):
            jj = j + b
            wait_gather(jj, b)
            multiply(jj, b)
            start_scatter(jj, b)
            ba = (b + 2) % 3
            wait_scatter(jj - 1, ba)
            start_gather(jj + 2, ba)

    for jj in (NCHUNK - 2, NCHUNK - 1):
        b = jj % 3
        wait_gather(jj, b)
        multiply(jj, b)
        start_scatter(jj, b)
    wait_scatter(NCHUNK - 3, (NCHUNK - 3) % 3)
    wait_scatter(NCHUNK - 2, (NCHUNK - 2) % 3)
    wait_scatter(NCHUNK - 1, (NCHUNK - 1) % 3)

    plsc.subcore_barrier()

    pltpu.sync_copy(acc.at[pl.ds(sid * ROWS_PT, ROWS_PT)],
                    out_hbm.at[cid, pl.ds(sid * ROWS_PT, ROWS_PT)])


_BLK = 2000


def _in_proj_body(x_ref, w_ref, b_ref, o_ref):
    o_ref[...] = lax.dot_general(
        x_ref[...], w_ref[...], (((1,), (1,)), ((), ())),
        preferred_element_type=jnp.float32) + b_ref[...]


def _in_proj(x, W, b):
    return pl.pallas_call(
        _in_proj_body,
        grid=(N // _BLK,),
        in_specs=[
            pl.BlockSpec((_BLK, D_FEAT), lambda i: (i, 0)),
            pl.BlockSpec((D_HID, D_FEAT), lambda i: (0, 0)),
            pl.BlockSpec((1, D_HID), lambda i: (0, 0)),
        ],
        out_specs=pl.BlockSpec((_BLK, D_HID), lambda i: (i, 0)),
        out_shape=jax.ShapeDtypeStruct((N, D_HID), jnp.float32),
    )(x, W, b)


NP = N // 2
PBLK = 1000


def _layer_body(p0_ref, p1_ref, h0_ref, w_ref, b_ref, o_ref, *, beta):
    support = p0_ref[0] + p1_ref[0] + ALPHA * h0_ref[...]
    z = lax.dot_general(support, w_ref[0], (((1,), (1,)), ((), ())),
                        preferred_element_type=jnp.float32) + b_ref[0]
    o_ref[...] = jnp.maximum((1.0 - beta) * support + beta * z, 0.0)


def _layer_tc(parts2, h02, Wd, b2, ell, beta):
    return pl.pallas_call(
        functools.partial(_layer_body, beta=beta),
        grid=(NP // PBLK,),
        in_specs=[
            pl.BlockSpec((1, PBLK, 2 * D_HID), lambda i: (0, i, 0)),
            pl.BlockSpec((1, PBLK, 2 * D_HID), lambda i: (1, i, 0)),
            pl.BlockSpec((PBLK, 2 * D_HID), lambda i: (i, 0)),
            pl.BlockSpec((1, 2 * D_HID, 2 * D_HID), lambda i: (ell, 0, 0)),
            pl.BlockSpec((1, 1, 2 * D_HID), lambda i: (ell, 0, 0)),
        ],
        out_specs=pl.BlockSpec((PBLK, 2 * D_HID), lambda i: (i, 0)),
        out_shape=jax.ShapeDtypeStruct((NP, 2 * D_HID), jnp.float32),
    )(parts2, parts2, h02, Wd, b2)


def _last_body(p0_ref, p1_ref, h0_ref, w_ref, b_ref, wo_ref, bo_ref, o_ref,
               *, beta):
    support = p0_ref[0] + p1_ref[0] + ALPHA * h0_ref[...]
    z = lax.dot_general(support, w_ref[0], (((1,), (1,)), ((), ())),
                        preferred_element_type=jnp.float32) + b_ref[0]
    h = jnp.maximum((1.0 - beta) * support + beta * z, 0.0)
    o_ref[...] = lax.dot_general(
        h, wo_ref[...], (((1,), (1,)), ((), ())),
        preferred_element_type=jnp.float32) + bo_ref[...]


def _last_tc(parts2, h02, Wd, b2, ell, Wo2, bo2, beta):
    return pl.pallas_call(
        functools.partial(_last_body, beta=beta),
        grid=(NP // PBLK,),
        in_specs=[
            pl.BlockSpec((1, PBLK, 2 * D_HID), lambda i: (0, i, 0)),
            pl.BlockSpec((1, PBLK, 2 * D_HID), lambda i: (1, i, 0)),
            pl.BlockSpec((PBLK, 2 * D_HID), lambda i: (i, 0)),
            pl.BlockSpec((1, 2 * D_HID, 2 * D_HID), lambda i: (ell, 0, 0)),
            pl.BlockSpec((1, 1, 2 * D_HID), lambda i: (ell, 0, 0)),
            pl.BlockSpec((2 * N_CLASSES, 2 * D_HID), lambda i: (0, 0)),
            pl.BlockSpec((1, 2 * N_CLASSES), lambda i: (0, 0)),
        ],
        out_specs=pl.BlockSpec((PBLK, 2 * N_CLASSES), lambda i: (i, 0)),
        out_shape=jax.ShapeDtypeStruct((NP, 2 * N_CLASSES), jnp.float32),
    )(parts2, parts2, h02, Wd, b2, Wo2, bo2)


def kernel(x, edge_index, edge_weight, W_in, b_in, Ws, bs, W_out, b_out):
    src = edge_index[0]
    dst3d = edge_index[1].reshape(NW, NCHUNK, CHUNK)

    Wd = jnp.zeros((N_LAYERS, 2 * D_HID, 2 * D_HID), jnp.float32)
    Wd = Wd.at[:, :D_HID, :D_HID].set(Ws).at[:, D_HID:, D_HID:].set(Ws)
    b2 = jnp.tile(bs, (1, 2)).reshape(N_LAYERS, 1, 2 * D_HID)
    Wo2 = jnp.zeros((2 * N_CLASSES, 2 * D_HID), jnp.float32)
    Wo2 = Wo2.at[:N_CLASSES, :D_HID].set(W_out).at[
        N_CLASSES:, D_HID:].set(W_out)
    bo2 = jnp.tile(b_out, 2).reshape(1, 2 * N_CLASSES)

    h = _in_proj(x, W_in, b_in.reshape(1, D_HID))
    h02 = h.reshape(NP, 2 * D_HID)
    h_sc = h
    out = None
    for ell in range(N_LAYERS):
        beta = float(np.log(0.5 / (ell + 1) + 1.0))
        parts = _spmm_sc(h_sc, src, dst3d, edge_weight)
        parts2 = parts.reshape(NC, NP, 2 * D_HID)
        if ell < N_LAYERS - 1:
            h2 = _layer_tc(parts2, h02, Wd, b2, ell, beta)
            h_sc = h2.reshape(N, D_HID)
        else:
            out2 = _last_tc(parts2, h02, Wd, b2, ell, Wo2, bo2, beta)
    return out2.reshape(N, N_CLASSES)
